# baseline (speedup 1.0000x reference)
import functools
import math

import jax
import jax.numpy as jnp
from jax import lax
from jax.experimental import pallas as pl
from jax.experimental.pallas import tpu as pltpu
from jax.experimental.pallas import tpu_sc as plsc

N = 10000
E = 160000
LATENT = 64
MP_STEPS = 2
SITL_STEPS = 2
DT = 0.002
SDT = DT / SITL_STEPS
SIGMA = 3.0 / (359.0 * math.pi)
PREF = 100.0
ETA_IJ = 2.0 * 0.01 * 0.01 / (0.01 + 0.01 + 1e-08)
GNN_INV_RADIUS = 1.0 / 3.0
N_TYPES = 9
TYPE_EMB_DIM = 16

NC = 2
NS = 16
NW = NC * NS
EPT = E // NW
G2 = EPT + 8
NGRP = G2 // 16
ROWS_A = 640
ROWS_B = N - 15 * ROWS_A

@functools.cache
def _mesh():
  return plsc.VectorSubcoreMesh(
      core_axis_name="c", subcore_axis_name="s", num_cores=NC, num_subcores=NS)


_f32 = jnp.float32
_i32 = jnp.int32


def _sc_sqrt(s):
  i = plsc.bitcast(s, _i32)
  y = plsc.bitcast(jnp.int32(0x1FBD1DF5) + lax.shift_right_arithmetic(i, 1),
                   _f32)
  y = 0.5 * (y + s / y)
  y = 0.5 * (y + s / y)
  y = 0.5 * (y + s / y)
  return y


def _zero_sh_1d(sid, sh, zb):
  @pl.when(sid < 15)
  def _():
    pltpu.sync_copy(zb, sh.at[pl.ds(sid * ROWS_A, ROWS_A)])

  @pl.when(sid == 15)
  def _():
    pltpu.sync_copy(zb.at[pl.ds(0, ROWS_B)], sh.at[pl.ds(15 * ROWS_A, ROWS_B)])


def _sph_body(xp, yp, zp, uxp, uyp, uzp, ish, jsh, rcv, snd,
              ax0, ay0, az0, ax1, ay1, az1, gx, gy, gz,
              xv, yv, zv, uxv, uyv, uzv, rhov, iv, jv, v0, v1, v2, zb,
              rho_sh, ax_sh, ay_sh, az_sh):
  cid = lax.axis_index("c")
  sid = lax.axis_index("s")
  wid = cid * NS + sid

  pltpu.sync_copy(xp, xv)
  pltpu.sync_copy(yp, yv)
  pltpu.sync_copy(zp, zv)
  pltpu.sync_copy(uxp, uxv)
  pltpu.sync_copy(uyp, uyv)
  pltpu.sync_copy(uzp, uzv)

  zi = jnp.full((16,), 0, _i32)
  zf = jnp.full((16,), 0.0, _f32)
  iv[pl.ds(G2 - 16, 16)] = zi
  jv[pl.ds(G2 - 16, 16)] = zi
  for t in range(ROWS_A // 16):
    zb[pl.ds(t * 16, 16)] = zf

  _zero_sh_1d(sid, rho_sh, zb)
  _zero_sh_1d(sid, ax_sh, zb)
  _zero_sh_1d(sid, ay_sh, zb)
  _zero_sh_1d(sid, az_sh, zb)
  plsc.subcore_barrier()

  iota = lax.iota(_i32, 16)

  ebase = wid * EPT
  pltpu.sync_copy(rcv.at[pl.ds(ebase, EPT)], iv.at[pl.ds(0, EPT)])
  pltpu.sync_copy(snd.at[pl.ds(ebase, EPT)], jv.at[pl.ds(0, EPT)])

  def _geom_grp(g, carry):
    off = pl.multiple_of(g * 16, 16)
    ii = iv[pl.ds(off, 16)]
    jj = jv[pl.ds(off, 16)]
    v0[pl.ds(off, 16)] = (plsc.load_gather(xv, [ii])
                          - plsc.load_gather(xv, [jj])) * GNN_INV_RADIUS
    v1[pl.ds(off, 16)] = (plsc.load_gather(yv, [ii])
                          - plsc.load_gather(yv, [jj])) * GNN_INV_RADIUS
    v2[pl.ds(off, 16)] = (plsc.load_gather(zv, [ii])
                          - plsc.load_gather(zv, [jj])) * GNN_INV_RADIUS
    return carry

  lax.fori_loop(0, NGRP, _geom_grp, 0)
  pltpu.sync_copy(v0.at[pl.ds(0, EPT)], gx.at[pl.ds(ebase, EPT)])
  pltpu.sync_copy(v1.at[pl.ds(0, EPT)], gy.at[pl.ds(ebase, EPT)])
  pltpu.sync_copy(v2.at[pl.ds(0, EPT)], gz.at[pl.ds(ebase, EPT)])

  for q in range(2):
    b1 = sid * (2 * EPT) + q * EPT
    pltpu.sync_copy(ish.at[pl.ds(b1, EPT)], iv.at[pl.ds(0, EPT)])
    pltpu.sync_copy(jsh.at[pl.ds(b1, EPT)], jv.at[pl.ds(0, EPT)])

    def _w_grp(g, carry):
      off = pl.multiple_of(g * 16, 16)
      ii = iv[pl.ds(off, 16)]
      jj = jv[pl.ds(off, 16)]
      dx = plsc.load_gather(xv, [ii]) - plsc.load_gather(xv, [jj])
      dy = plsc.load_gather(yv, [ii]) - plsc.load_gather(yv, [jj])
      dz = plsc.load_gather(zv, [ii]) - plsc.load_gather(zv, [jj])
      d = _sc_sqrt(dx * dx + dy * dy + dz * dz + 1e-16)
      q1 = jnp.maximum(0.0, 1.0 - d)
      q2 = jnp.maximum(0.0, 2.0 - d)
      q3 = jnp.maximum(0.0, 3.0 - d)
      q1 = q1 * q1 * q1 * q1 * q1
      q2 = q2 * q2 * q2 * q2 * q2
      q3 = q3 * q3 * q3 * q3 * q3
      w = SIGMA * (q3 - 6.0 * q2 + 15.0 * q1)
      w = jnp.where(off + iota < EPT, w, 0.0)
      v0[pl.ds(off, 16)] = w
      return carry

    lax.fori_loop(0, NGRP, _w_grp, 0)
    pltpu.sync_copy(v0, rho_sh.at[iv], add=True)

  plsc.subcore_barrier()
  pltpu.sync_copy(rho_sh, rhov)

  b2 = wid * EPT
  pltpu.sync_copy(ish.at[pl.ds(b2, EPT)], iv.at[pl.ds(0, EPT)])
  pltpu.sync_copy(jsh.at[pl.ds(b2, EPT)], jv.at[pl.ds(0, EPT)])

  def _a_grp(g, carry):
    off = pl.multiple_of(g * 16, 16)
    ii = iv[pl.ds(off, 16)]
    jj = jv[pl.ds(off, 16)]
    dx = plsc.load_gather(xv, [ii]) - plsc.load_gather(xv, [jj])
    dy = plsc.load_gather(yv, [ii]) - plsc.load_gather(yv, [jj])
    dz = plsc.load_gather(zv, [ii]) - plsc.load_gather(zv, [jj])
    dux = plsc.load_gather(uxv, [ii]) - plsc.load_gather(uxv, [jj])
    duy = plsc.load_gather(uyv, [ii]) - plsc.load_gather(uyv, [jj])
    duz = plsc.load_gather(uzv, [ii]) - plsc.load_gather(uzv, [jj])
    ri = plsc.load_gather(rhov, [ii])
    rj = plsc.load_gather(rhov, [jj])
    d = _sc_sqrt(dx * dx + dy * dy + dz * dz + 1e-16)
    q1 = jnp.maximum(0.0, 1.0 - d)
    q2 = jnp.maximum(0.0, 2.0 - d)
    q3 = jnp.maximum(0.0, 3.0 - d)
    q1 = q1 * q1 * q1 * q1
    q2 = q2 * q2 * q2 * q2
    q3 = q3 * q3 * q3 * q3
    gw = (-5.0 * SIGMA) * (q3 - 6.0 * q2 + 15.0 * q1)
    p_i = PREF * (ri - 1.0)
    p_j = PREF * (rj - 1.0)
    p_ij = (rj * p_i + ri * p_j) / (ri + rj)
    wv = 1.0 / (ri * ri) + 1.0 / (rj * rj)
    c = wv * gw / (d + 1e-08)
    msk = off + iota < EPT
    v0[pl.ds(off, 16)] = jnp.where(
        msk, c * (-p_ij * dx + ETA_IJ * dux), 0.0)
    v1[pl.ds(off, 16)] = jnp.where(
        msk, c * (-p_ij * dy + ETA_IJ * duy), 0.0)
    v2[pl.ds(off, 16)] = jnp.where(
        msk, c * (-p_ij * dz + ETA_IJ * duz), 0.0)
    return carry

  lax.fori_loop(0, NGRP, _a_grp, 0)
  pltpu.sync_copy(v0, ax_sh.at[iv], add=True)
  pltpu.sync_copy(v1, ay_sh.at[iv], add=True)
  pltpu.sync_copy(v2, az_sh.at[iv], add=True)
  plsc.subcore_barrier()

  for sh, o0, o1 in ((ax_sh, ax0, ax1), (ay_sh, ay0, ay1), (az_sh, az0, az1)):
    for core, out in ((0, o0), (1, o1)):
      @pl.when((cid == core) & (sid < 15))
      def _():
        r0 = sid * ROWS_A
        pltpu.sync_copy(sh.at[pl.ds(r0, ROWS_A)], out.at[pl.ds(r0, ROWS_A)])

      @pl.when((cid == core) & (sid == 15))
      def _():
        pltpu.sync_copy(sh.at[pl.ds(15 * ROWS_A, ROWS_B)],
                        zb.at[pl.ds(0, ROWS_B)])
        pltpu.sync_copy(zb.at[pl.ds(0, ROWS_B)],
                        out.at[pl.ds(15 * ROWS_A, ROWS_B)])


@functools.cache
def _sph_sc_built():
  return pl.kernel(
      _sph_body,
      out_type=tuple([jax.ShapeDtypeStruct((N,), _f32)] * 6
                     + [jax.ShapeDtypeStruct((E,), _f32)] * 3),
      mesh=_mesh(),
      compiler_params=pltpu.CompilerParams(needs_layout_passes=False),
      scratch_types=(
          [pltpu.VMEM((N,), _f32)] * 7
          + [pltpu.VMEM((G2,), _i32)] * 2
          + [pltpu.VMEM((G2,), _f32)] * 3
          + [pltpu.VMEM((ROWS_A,), _f32)]
          + [pltpu.VMEM_SHARED((N,), _f32)] * 4),
  )


def _sph_sc(*args):
  return _sph_sc_built()(*args)


_GC = 200
L2 = 2 * LATENT


def _gather_body(gk_t, rcv, snd, out_r, out_s, ivr, ivs, rows_a, rows_b,
                 sem_a, sem_b):
  cid = lax.axis_index("c")
  sid = lax.axis_index("s")
  wid = cid * NS + sid
  base = wid * EPT
  pltpu.sync_copy(rcv.at[pl.ds(base, EPT)], ivr)
  pltpu.sync_copy(snd.at[pl.ds(base, EPT)], ivs)

  n_units = 2 * (EPT // _GC)
  bufs = (rows_a, rows_b)
  sems = (sem_a, sem_b)

  def _src(u):
    ch = u // 2
    idx = ivr if u % 2 == 0 else ivs
    return gk_t.at[idx.at[pl.ds(ch * _GC, _GC)]]

  def _dst(u):
    ch = u // 2
    out = out_r if u % 2 == 0 else out_s
    return out.at[pl.ds(base + ch * _GC, _GC)]

  descs = [None] * n_units
  descs[0] = pltpu.async_copy(_src(0), bufs[0], sems[0])
  for u in range(n_units):
    if u + 1 < n_units:
      descs[u + 1] = pltpu.async_copy(_src(u + 1), bufs[(u + 1) % 2],
                                      sems[(u + 1) % 2])
    descs[u].wait()
    pltpu.sync_copy(bufs[u % 2], _dst(u))


@functools.cache
def _gather_sc_built():
  return pl.kernel(
      _gather_body,
      out_type=(jax.ShapeDtypeStruct((E, L2), _f32),
                jax.ShapeDtypeStruct((E, L2), _f32)),
      mesh=_mesh(),
      compiler_params=pltpu.CompilerParams(needs_layout_passes=False),
      scratch_types=(
          [pltpu.VMEM((EPT,), _i32)] * 2
          + [pltpu.VMEM((_GC, L2), _f32)] * 2
          + [pltpu.SemaphoreType.DMA] * 2),
  )


def _gather_sc(*args):
  return _gather_sc_built()(*args)


_SC_CH = 192
_SC_TAIL = EPT - (EPT // _SC_CH) * _SC_CH


def _scatter_body(e_t, rcv, zer, aggp, idx, idxt, rows, agg_sh):
  cid = lax.axis_index("c")
  sid = lax.axis_index("s")
  wid = cid * NS + sid

  @pl.when(sid < 15)
  def _():
    r0 = sid * ROWS_A
    pltpu.sync_copy(zer.at[pl.ds(r0, ROWS_A)], agg_sh.at[pl.ds(r0, ROWS_A)])

  @pl.when(sid == 15)
  def _():
    pltpu.sync_copy(zer.at[pl.ds(15 * ROWS_A, ROWS_B)],
                    agg_sh.at[pl.ds(15 * ROWS_A, ROWS_B)])

  plsc.subcore_barrier()

  for ch in range(EPT // _SC_CH):
    eb = wid * EPT + ch * _SC_CH
    pltpu.sync_copy(rcv.at[pl.ds(eb, _SC_CH)], idx)
    pltpu.sync_copy(e_t.at[pl.ds(eb, _SC_CH)], rows)
    pltpu.sync_copy(rows, agg_sh.at[idx], add=True)

  tb = wid * EPT + (EPT // _SC_CH) * _SC_CH
  pltpu.sync_copy(rcv.at[pl.ds(tb, _SC_TAIL)], idxt)
  pltpu.sync_copy(e_t.at[pl.ds(tb, _SC_TAIL)], rows.at[pl.ds(0, _SC_TAIL)])
  pltpu.sync_copy(rows.at[pl.ds(0, _SC_TAIL)], agg_sh.at[idxt], add=True)

  plsc.subcore_barrier()

  @pl.when(sid < 15)
  def _():
    r0 = sid * ROWS_A
    pltpu.sync_copy(agg_sh.at[pl.ds(r0, ROWS_A)], aggp.at[cid, pl.ds(r0, ROWS_A)])

  @pl.when(sid == 15)
  def _():
    pltpu.sync_copy(agg_sh.at[pl.ds(15 * ROWS_A, ROWS_B)],
                    aggp.at[cid, pl.ds(15 * ROWS_A, ROWS_B)])


@functools.cache
def _scatter_sc_built():
  return pl.kernel(
      _scatter_body,
      out_type=jax.ShapeDtypeStruct((NC, N, L2), _f32),
      mesh=_mesh(),
      compiler_params=pltpu.CompilerParams(needs_layout_passes=False),
      scratch_types=(
          [pltpu.VMEM((_SC_CH,), _i32),
           pltpu.VMEM((_SC_TAIL,), _i32),
           pltpu.VMEM((_SC_CH, L2), _f32),
           pltpu.VMEM_SHARED((N, L2), _f32)]),
  )


def _scatter_sc(*args):
  return _scatter_sc_built()(*args)


_BN = 2000
_BE = 6400


def _ln(x):
  m = jnp.mean(x, axis=1, keepdims=True)
  x0 = x - m
  v = jnp.mean(x0 * x0, axis=1, keepdims=True)
  return x0 / jnp.sqrt(v + 1e-6)


def _full(shape):
  return pl.BlockSpec(shape, lambda i: (0,) * len(shape))


def _rows(block):
  return pl.BlockSpec(block, lambda i: (i,) + (0,) * (len(block) - 1))


def _node_enc_body(u_r, tag_r, emb_r, we1a_r, we1b_r, be1_r, we2_r, be2_r,
                   w1b_r, w1c_r, b1e_r, h_o, gk_o):
  vel = u_r[...] * DT
  x1 = (vel[:, 0:1] * we1a_r[0:1, :] + vel[:, 1:2] * we1a_r[1:2, :]
        + vel[:, 2:3] * we1a_r[2:3, :])
  m_t = jnp.dot(emb_r[...], we1b_r[...], preferred_element_type=_f32)
  onehot = (tag_r[...] ==
            lax.broadcasted_iota(_i32, (1, N_TYPES), 1).astype(_f32))
  x2 = jnp.dot(onehot.astype(_f32), m_t, preferred_element_type=_f32)
  z = jax.nn.relu(x1 + x2 + be1_r[...])
  h = _ln(jnp.dot(z, we2_r[...], preferred_element_type=_f32) + be2_r[...])
  h_o[...] = h
  gk_o[...] = jnp.concatenate(
      [jnp.dot(h, w1b_r[...], preferred_element_type=_f32) + b1e_r[...],
       jnp.dot(h, w1c_r[...], preferred_element_type=_f32)], axis=1)


def _node_enc(u_t, tag_f, emb, we1a, we1b, be1, we2, be2, w1b, w1c, b1e):
  return pl.pallas_call(
      _node_enc_body,
      grid=(N // _BN,),
      in_specs=[_rows((_BN, 3)), _rows((_BN, 1)), _full((N_TYPES, TYPE_EMB_DIM)),
                _full((3, LATENT)), _full((TYPE_EMB_DIM, LATENT)),
                _full((1, LATENT)), _full((LATENT, LATENT)), _full((1, LATENT)),
                _full((LATENT, LATENT)), _full((LATENT, LATENT)),
                _full((1, LATENT))],
      out_specs=[_rows((_BN, LATENT)), _rows((_BN, L2))],
      out_shape=[jax.ShapeDtypeStruct((N, LATENT), _f32),
                 jax.ShapeDtypeStruct((N, L2), _f32)],
  )(u_t, tag_f, emb, we1a, we1b, be1, we2, be2, w1b, w1c, b1e)


def _edge_enc_body(dx_r, dy_r, dz_r, wg1_r, bg1_r, wg2_r, bg2_r, e_o):
  dx, dy, dz = dx_r[...], dy_r[...], dz_r[...]
  dist = jnp.sqrt(dx * dx + dy * dy + dz * dz + 1e-16)
  feat = (dx * wg1_r[0:1, :] + dy * wg1_r[1:2, :] + dz * wg1_r[2:3, :]
          + dist * wg1_r[3:4, :] + bg1_r[...])
  e = _ln(jnp.dot(jax.nn.relu(feat), wg2_r[...],
                  preferred_element_type=_f32) + bg2_r[...])
  e_o[...] = jnp.concatenate([e, jnp.zeros_like(e)], axis=1)


def _edge_enc(dx, dy, dz, wg1, bg1, wg2, bg2):
  return pl.pallas_call(
      _edge_enc_body,
      grid=(E // _BE,),
      in_specs=[_rows((_BE, 1))] * 3
      + [_full((4, LATENT)), _full((1, LATENT)), _full((LATENT, LATENT)),
         _full((1, LATENT))],
      out_specs=_rows((_BE, L2)),
      out_shape=jax.ShapeDtypeStruct((E, L2), _f32),
  )(dx, dy, dz, wg1, bg1, wg2, bg2)


def _edge_mlp_body(e_r, hr_r, ks_r, w1a_r, w2_r, b2_r, e_o):
  e = e_r[:, :LATENT]
  t = jax.nn.relu(
      jnp.dot(e, w1a_r[...], preferred_element_type=_f32)
      + hr_r[:, :LATENT] + ks_r[:, LATENT:])
  en = e + _ln(jnp.dot(t, w2_r[...], preferred_element_type=_f32) + b2_r[...])
  e_o[...] = jnp.concatenate([en, jnp.zeros_like(en)], axis=1)


def _edge_mlp(e, h_r, k_s, w1a, w2, b2):
  return pl.pallas_call(
      _edge_mlp_body,
      grid=(E // _BE,),
      in_specs=[_rows((_BE, L2))] * 3
      + [_full((LATENT, LATENT)), _full((LATENT, LATENT)), _full((1, LATENT))],
      out_specs=_rows((_BE, L2)),
      out_shape=jax.ShapeDtypeStruct((E, L2), _f32),
  )(e, h_r, k_s, w1a, w2, b2)


def _node_mlp_prep_body(h_r, a0_r, a1_r, wn1a_r, wn1b_r, bn1_r, wn2_r, bn2_r,
                        w1b_r, w1c_r, b1e_r, h_o, gk_o):
  agg = a0_r[:, :LATENT] + a1_r[:, :LATENT]
  z = jax.nn.relu(
      jnp.dot(h_r[...], wn1a_r[...], preferred_element_type=_f32)
      + jnp.dot(agg, wn1b_r[...], preferred_element_type=_f32) + bn1_r[...])
  hn = h_r[...] + _ln(
      jnp.dot(z, wn2_r[...], preferred_element_type=_f32) + bn2_r[...])
  h_o[...] = hn
  gk_o[...] = jnp.concatenate(
      [jnp.dot(hn, w1b_r[...], preferred_element_type=_f32) + b1e_r[...],
       jnp.dot(hn, w1c_r[...], preferred_element_type=_f32)], axis=1)


def _node_mlp_prep(h, a0, a1, wn1a, wn1b, bn1, wn2, bn2, w1b, w1c, b1e):
  return pl.pallas_call(
      _node_mlp_prep_body,
      grid=(N // _BN,),
      in_specs=[_rows((_BN, LATENT))] + [_rows((_BN, L2))] * 2
      + [_full((LATENT, LATENT))] * 2 + [_full((1, LATENT))]
      + [_full((LATENT, LATENT)), _full((1, LATENT))]
      + [_full((LATENT, LATENT))] * 2 + [_full((1, LATENT))],
      out_specs=[_rows((_BN, LATENT)), _rows((_BN, L2))],
      out_shape=[jax.ShapeDtypeStruct((N, LATENT), _f32),
                 jax.ShapeDtypeStruct((N, L2), _f32)],
  )(h, a0, a1, wn1a, wn1b, bn1, wn2, bn2, w1b, w1c, b1e)


def _node_mlp_body(h_r, a0_r, a1_r, wn1a_r, wn1b_r, bn1_r, wn2_r, bn2_r, h_o):
  agg = a0_r[:, :LATENT] + a1_r[:, :LATENT]
  z = jax.nn.relu(
      jnp.dot(h_r[...], wn1a_r[...], preferred_element_type=_f32)
      + jnp.dot(agg, wn1b_r[...], preferred_element_type=_f32) + bn1_r[...])
  h_o[...] = h_r[...] + _ln(
      jnp.dot(z, wn2_r[...], preferred_element_type=_f32) + bn2_r[...])


def _node_mlp(h, a0, a1, wn1a, wn1b, bn1, wn2, bn2):
  return pl.pallas_call(
      _node_mlp_body,
      grid=(N // _BN,),
      in_specs=[_rows((_BN, LATENT))] + [_rows((_BN, L2))] * 2
      + [_full((LATENT, LATENT))] * 2 + [_full((1, LATENT))]
      + [_full((LATENT, LATENT)), _full((1, LATENT))],
      out_specs=_rows((_BN, LATENT)),
      out_shape=jax.ShapeDtypeStruct((N, LATENT), _f32),
  )(h, a0, a1, wn1a, wn1b, bn1, wn2, bn2)


def _decoder_body(h_r, wd1_r, bd1_r, wd2_r, bd2_r, d_o):
  t = jax.nn.relu(
      jnp.dot(h_r[...], wd1_r[...], preferred_element_type=_f32) + bd1_r[...])
  d_o[...] = jnp.dot(t, wd2_r[...], preferred_element_type=_f32) + bd2_r[...]


def _decoder(h, wd1, bd1, wd2, bd2):
  return pl.pallas_call(
      _decoder_body,
      grid=(N // _BN,),
      in_specs=[_rows((_BN, LATENT)), _full((LATENT, LATENT)),
                _full((1, LATENT)), _full((LATENT, 3)), _full((1, 3))],
      out_specs=_rows((_BN, 3)),
      out_shape=jax.ShapeDtypeStruct((N, 3), _f32),
  )(h, wd1, bd1, wd2, bd2)


def _integrate_body(u_r, r_r, a0_r, a1_r, d_r, r0_r, u0_r, un_o, rn_o, ans_o):
  acc = a0_r[...] + a1_r[...] + d_r[...] * (1.0 / (DT * DT))
  un = u_r[...] + SDT * acc
  rn = r_r[...] + SDT * un
  un_o[...] = un
  rn_o[...] = rn
  ans_o[...] = (rn - r0_r[...]) - u0_r[...] * DT


def _integrate(u3, r3, a0, a1, d_t, r03, u03):
  return pl.pallas_call(
      _integrate_body,
      out_shape=[jax.ShapeDtypeStruct((3, N), _f32)] * 3,
  )(u3, r3, a0, a1, d_t, r03, u03)


def kernel(abs_pos, vel_hist, tag, sph_edge_index, gnn_edge_index, params):
  r3 = abs_pos[:, -1, :].T
  u3 = vel_hist.T * (1.0 / DT)
  ish = sph_edge_index[0]
  jsh = sph_edge_index[1]
  rcv = gnn_edge_index[0]
  snd = gnn_edge_index[1]
  tag_f = tag.astype(_f32)[:, None]

  (we1, be1), (we2, be2) = params["node_enc"]
  emb = params["type_emb"]
  (wg1, bg1), (wg2, bg2) = params["edge_enc"]
  (wd1, bd1), (wd2, bd2) = params["decoder"]
  be1r, be2r = be1[None, :], be2[None, :]
  bg1r, bg2r = bg1[None, :], bg2[None, :]
  bd1r, bd2r = bd1[None, :], bd2[None, :]

  edge_w = []
  for m in range(MP_STEPS):
    (w1, b1), (w2, b2) = params["edge_mlps"][m]
    edge_w.append((w1[:LATENT], w1[LATENT:2 * LATENT], w1[2 * LATENT:],
                   b1[None, :], w2, b2[None, :]))
  node_w = []
  for m in range(MP_STEPS):
    (wn1, bn1), (wn2, bn2) = params["node_mlps"][m]
    node_w.append((wn1[:LATENT], wn1[LATENT:], bn1[None, :], wn2,
                   bn2[None, :]))

  r03, u03 = r3, u3
  zer = jnp.zeros((N, L2), _f32)
  ans = None
  for _ in range(SITL_STEPS):
    (ax0, ay0, az0, ax1, ay1, az1, gx, gy, gz) = _sph_sc(
        r3[0], r3[1], r3[2], u3[0], u3[1], u3[2], ish, jsh, rcv, snd)
    a0 = jnp.stack([ax0, ay0, az0])
    a1 = jnp.stack([ax1, ay1, az1])
    dx = gx[:, None]
    dy = gy[:, None]
    dz = gz[:, None]

    w1a0, w1b0, w1c0, b1e0, _, _ = edge_w[0]
    h, gk_t = _node_enc(u3.T, tag_f, emb, we1[:3], we1[3:], be1r, we2,
                        be2r, w1b0, w1c0, b1e0)
    e = _edge_enc(dx, dy, dz, wg1, bg1r, wg2, bg2r)

    for m in range(MP_STEPS):
      w1a, _, _, _, w2, b2r = edge_w[m]
      h_r, k_s = _gather_sc(gk_t, rcv, snd)
      e = _edge_mlp(e, h_r, k_s, w1a, w2, b2r)
      aggp = _scatter_sc(e, rcv, zer)
      wn1a, wn1b, bn1r, wn2, bn2r = node_w[m]
      if m + 1 < MP_STEPS:
        _, w1b_n, w1c_n, b1e_n, _, _ = edge_w[m + 1]
        h, gk_t = _node_mlp_prep(h, aggp[0], aggp[1], wn1a, wn1b, bn1r,
                                 wn2, bn2r, w1b_n, w1c_n, b1e_n)
      else:
        h = _node_mlp(h, aggp[0], aggp[1], wn1a, wn1b, bn1r, wn2, bn2r)

    dcd = _decoder(h, wd1, bd1r, wd2, bd2r)
    u3, r3, ans = _integrate(u3, r3, a0, a1, dcd.T, r03, u03)

  return ans.T


# speedup vs baseline: 1.0347x; 1.0347x over previous
import functools
import math

import jax
import jax.numpy as jnp
from jax import lax
from jax.experimental import pallas as pl
from jax.experimental.pallas import tpu as pltpu
from jax.experimental.pallas import tpu_sc as plsc

N = 10000
E = 160000
LATENT = 64
MP_STEPS = 2
SITL_STEPS = 2
DT = 0.002
SDT = DT / SITL_STEPS
SIGMA = 3.0 / (359.0 * math.pi)
PREF = 100.0
ETA_IJ = 2.0 * 0.01 * 0.01 / (0.01 + 0.01 + 1e-08)
GNN_INV_RADIUS = 1.0 / 3.0
N_TYPES = 9
TYPE_EMB_DIM = 16

NC = 2
NS = 16
NW = NC * NS
EPT = E // NW
G2 = EPT + 8
NGRP = G2 // 16
ROWS_A = 640
ROWS_B = N - 15 * ROWS_A

@functools.cache
def _mesh():
  return plsc.VectorSubcoreMesh(
      core_axis_name="c", subcore_axis_name="s", num_cores=NC, num_subcores=NS)


_f32 = jnp.float32
_i32 = jnp.int32


def _sc_sqrt(s):
  i = plsc.bitcast(s, _i32)
  y = plsc.bitcast(jnp.int32(0x1FBD1DF5) + lax.shift_right_arithmetic(i, 1),
                   _f32)
  y = 0.5 * (y + s / y)
  y = 0.5 * (y + s / y)
  y = 0.5 * (y + s / y)
  return y


def _zero_sh_1d(sid, sh, zb):
  @pl.when(sid < 15)
  def _():
    pltpu.sync_copy(zb, sh.at[pl.ds(sid * ROWS_A, ROWS_A)])

  @pl.when(sid == 15)
  def _():
    pltpu.sync_copy(zb.at[pl.ds(0, ROWS_B)], sh.at[pl.ds(15 * ROWS_A, ROWS_B)])


def _sph_body(xp, yp, zp, uxp, uyp, uzp, ish, jsh, rcv, snd,
              ax0, ay0, az0, ax1, ay1, az1, gx, gy, gz,
              xv, yv, zv, uxv, uyv, uzv, rhov, iv, jv, v0, v1, v2, zb,
              rho_sh, ax_sh, ay_sh, az_sh):
  cid = lax.axis_index("c")
  sid = lax.axis_index("s")
  wid = cid * NS + sid

  pltpu.sync_copy(xp, xv)
  pltpu.sync_copy(yp, yv)
  pltpu.sync_copy(zp, zv)
  pltpu.sync_copy(uxp, uxv)
  pltpu.sync_copy(uyp, uyv)
  pltpu.sync_copy(uzp, uzv)

  zi = jnp.full((16,), 0, _i32)
  zf = jnp.full((16,), 0.0, _f32)
  iv[pl.ds(G2 - 16, 16)] = zi
  jv[pl.ds(G2 - 16, 16)] = zi
  for t in range(ROWS_A // 16):
    zb[pl.ds(t * 16, 16)] = zf

  _zero_sh_1d(sid, rho_sh, zb)
  _zero_sh_1d(sid, ax_sh, zb)
  _zero_sh_1d(sid, ay_sh, zb)
  _zero_sh_1d(sid, az_sh, zb)
  plsc.subcore_barrier()

  iota = lax.iota(_i32, 16)

  ebase = wid * EPT
  pltpu.sync_copy(rcv.at[pl.ds(ebase, EPT)], iv.at[pl.ds(0, EPT)])
  pltpu.sync_copy(snd.at[pl.ds(ebase, EPT)], jv.at[pl.ds(0, EPT)])

  def _geom_grp(g, carry):
    off = pl.multiple_of(g * 16, 16)
    ii = iv[pl.ds(off, 16)]
    jj = jv[pl.ds(off, 16)]
    v0[pl.ds(off, 16)] = (plsc.load_gather(xv, [ii])
                          - plsc.load_gather(xv, [jj])) * GNN_INV_RADIUS
    v1[pl.ds(off, 16)] = (plsc.load_gather(yv, [ii])
                          - plsc.load_gather(yv, [jj])) * GNN_INV_RADIUS
    v2[pl.ds(off, 16)] = (plsc.load_gather(zv, [ii])
                          - plsc.load_gather(zv, [jj])) * GNN_INV_RADIUS
    return carry

  lax.fori_loop(0, NGRP, _geom_grp, 0)
  pltpu.sync_copy(v0.at[pl.ds(0, EPT)], gx.at[pl.ds(ebase, EPT)])
  pltpu.sync_copy(v1.at[pl.ds(0, EPT)], gy.at[pl.ds(ebase, EPT)])
  pltpu.sync_copy(v2.at[pl.ds(0, EPT)], gz.at[pl.ds(ebase, EPT)])

  for q in range(2):
    b1 = sid * (2 * EPT) + q * EPT
    pltpu.sync_copy(ish.at[pl.ds(b1, EPT)], iv.at[pl.ds(0, EPT)])
    pltpu.sync_copy(jsh.at[pl.ds(b1, EPT)], jv.at[pl.ds(0, EPT)])

    def _w_grp(g, carry):
      off = pl.multiple_of(g * 16, 16)
      ii = iv[pl.ds(off, 16)]
      jj = jv[pl.ds(off, 16)]
      dx = plsc.load_gather(xv, [ii]) - plsc.load_gather(xv, [jj])
      dy = plsc.load_gather(yv, [ii]) - plsc.load_gather(yv, [jj])
      dz = plsc.load_gather(zv, [ii]) - plsc.load_gather(zv, [jj])
      d = _sc_sqrt(dx * dx + dy * dy + dz * dz + 1e-16)
      q1 = jnp.maximum(0.0, 1.0 - d)
      q2 = jnp.maximum(0.0, 2.0 - d)
      q3 = jnp.maximum(0.0, 3.0 - d)
      q1 = q1 * q1 * q1 * q1 * q1
      q2 = q2 * q2 * q2 * q2 * q2
      q3 = q3 * q3 * q3 * q3 * q3
      w = SIGMA * (q3 - 6.0 * q2 + 15.0 * q1)
      w = jnp.where(off + iota < EPT, w, 0.0)
      v0[pl.ds(off, 16)] = w
      return carry

    lax.fori_loop(0, NGRP, _w_grp, 0)
    pltpu.sync_copy(v0, rho_sh.at[iv], add=True)

  plsc.subcore_barrier()
  pltpu.sync_copy(rho_sh, rhov)

  b2 = wid * EPT
  pltpu.sync_copy(ish.at[pl.ds(b2, EPT)], iv.at[pl.ds(0, EPT)])
  pltpu.sync_copy(jsh.at[pl.ds(b2, EPT)], jv.at[pl.ds(0, EPT)])

  def _a_grp(g, carry):
    off = pl.multiple_of(g * 16, 16)
    ii = iv[pl.ds(off, 16)]
    jj = jv[pl.ds(off, 16)]
    dx = plsc.load_gather(xv, [ii]) - plsc.load_gather(xv, [jj])
    dy = plsc.load_gather(yv, [ii]) - plsc.load_gather(yv, [jj])
    dz = plsc.load_gather(zv, [ii]) - plsc.load_gather(zv, [jj])
    dux = plsc.load_gather(uxv, [ii]) - plsc.load_gather(uxv, [jj])
    duy = plsc.load_gather(uyv, [ii]) - plsc.load_gather(uyv, [jj])
    duz = plsc.load_gather(uzv, [ii]) - plsc.load_gather(uzv, [jj])
    ri = plsc.load_gather(rhov, [ii])
    rj = plsc.load_gather(rhov, [jj])
    d = _sc_sqrt(dx * dx + dy * dy + dz * dz + 1e-16)
    q1 = jnp.maximum(0.0, 1.0 - d)
    q2 = jnp.maximum(0.0, 2.0 - d)
    q3 = jnp.maximum(0.0, 3.0 - d)
    q1 = q1 * q1 * q1 * q1
    q2 = q2 * q2 * q2 * q2
    q3 = q3 * q3 * q3 * q3
    gw = (-5.0 * SIGMA) * (q3 - 6.0 * q2 + 15.0 * q1)
    p_i = PREF * (ri - 1.0)
    p_j = PREF * (rj - 1.0)
    p_ij = (rj * p_i + ri * p_j) / (ri + rj)
    wv = 1.0 / (ri * ri) + 1.0 / (rj * rj)
    c = wv * gw / (d + 1e-08)
    msk = off + iota < EPT
    v0[pl.ds(off, 16)] = jnp.where(
        msk, c * (-p_ij * dx + ETA_IJ * dux), 0.0)
    v1[pl.ds(off, 16)] = jnp.where(
        msk, c * (-p_ij * dy + ETA_IJ * duy), 0.0)
    v2[pl.ds(off, 16)] = jnp.where(
        msk, c * (-p_ij * dz + ETA_IJ * duz), 0.0)
    return carry

  lax.fori_loop(0, NGRP, _a_grp, 0)
  pltpu.sync_copy(v0, ax_sh.at[iv], add=True)
  pltpu.sync_copy(v1, ay_sh.at[iv], add=True)
  pltpu.sync_copy(v2, az_sh.at[iv], add=True)
  plsc.subcore_barrier()

  for sh, o0, o1 in ((ax_sh, ax0, ax1), (ay_sh, ay0, ay1), (az_sh, az0, az1)):
    for core, out in ((0, o0), (1, o1)):
      @pl.when((cid == core) & (sid < 15))
      def _():
        r0 = sid * ROWS_A
        pltpu.sync_copy(sh.at[pl.ds(r0, ROWS_A)], out.at[pl.ds(r0, ROWS_A)])

      @pl.when((cid == core) & (sid == 15))
      def _():
        pltpu.sync_copy(sh.at[pl.ds(15 * ROWS_A, ROWS_B)],
                        zb.at[pl.ds(0, ROWS_B)])
        pltpu.sync_copy(zb.at[pl.ds(0, ROWS_B)],
                        out.at[pl.ds(15 * ROWS_A, ROWS_B)])


@functools.cache
def _sph_sc_built():
  return pl.kernel(
      _sph_body,
      out_type=tuple([jax.ShapeDtypeStruct((N,), _f32)] * 6
                     + [jax.ShapeDtypeStruct((E,), _f32)] * 3),
      mesh=_mesh(),
      compiler_params=pltpu.CompilerParams(needs_layout_passes=False),
      scratch_types=(
          [pltpu.VMEM((N,), _f32)] * 7
          + [pltpu.VMEM((G2,), _i32)] * 2
          + [pltpu.VMEM((G2,), _f32)] * 3
          + [pltpu.VMEM((ROWS_A,), _f32)]
          + [pltpu.VMEM_SHARED((N,), _f32)] * 4),
  )


def _sph_sc(*args):
  return _sph_sc_built()(*args)


_GC = 1000
L2 = 2 * LATENT
_bf16 = jnp.bfloat16


def _gather_body(gk_t, rcv, snd, out_r, out_s, idxr, idxs, rows):
  cid = lax.axis_index("c")
  sid = lax.axis_index("s")
  wid = cid * NS + sid
  for ch in range(EPT // _GC):
    eb = wid * EPT + ch * _GC
    pltpu.sync_copy(rcv.at[pl.ds(eb, _GC)], idxr)
    pltpu.sync_copy(snd.at[pl.ds(eb, _GC)], idxs)
    pltpu.sync_copy(gk_t.at[idxr], rows)
    pltpu.sync_copy(rows, out_r.at[pl.ds(eb, _GC)])
    pltpu.sync_copy(gk_t.at[idxs], rows)
    pltpu.sync_copy(rows, out_s.at[pl.ds(eb, _GC)])


@functools.cache
def _gather_sc_built():
  return pl.kernel(
      _gather_body,
      out_type=(jax.ShapeDtypeStruct((E, L2), _f32),
                jax.ShapeDtypeStruct((E, L2), _f32)),
      mesh=_mesh(),
      compiler_params=pltpu.CompilerParams(needs_layout_passes=False),
      scratch_types=(
          [pltpu.VMEM((_GC,), _i32)] * 2
          + [pltpu.VMEM((_GC, L2), _f32)]),
  )


def _gather_sc(*args):
  return _gather_sc_built()(*args)


_SC_CH = 192
_SC_TAIL = EPT - (EPT // _SC_CH) * _SC_CH


def _scatter_body(e_t, rcv, zer, aggp, idx, idxt, rows, agg_sh):
  cid = lax.axis_index("c")
  sid = lax.axis_index("s")
  wid = cid * NS + sid

  @pl.when(sid < 15)
  def _():
    r0 = sid * ROWS_A
    pltpu.sync_copy(zer.at[pl.ds(r0, ROWS_A)], agg_sh.at[pl.ds(r0, ROWS_A)])

  @pl.when(sid == 15)
  def _():
    pltpu.sync_copy(zer.at[pl.ds(15 * ROWS_A, ROWS_B)],
                    agg_sh.at[pl.ds(15 * ROWS_A, ROWS_B)])

  plsc.subcore_barrier()

  for ch in range(EPT // _SC_CH):
    eb = wid * EPT + ch * _SC_CH
    pltpu.sync_copy(rcv.at[pl.ds(eb, _SC_CH)], idx)
    pltpu.sync_copy(e_t.at[pl.ds(eb, _SC_CH)], rows)
    pltpu.sync_copy(rows, agg_sh.at[idx], add=True)

  tb = wid * EPT + (EPT // _SC_CH) * _SC_CH
  pltpu.sync_copy(rcv.at[pl.ds(tb, _SC_TAIL)], idxt)
  pltpu.sync_copy(e_t.at[pl.ds(tb, _SC_TAIL)], rows.at[pl.ds(0, _SC_TAIL)])
  pltpu.sync_copy(rows.at[pl.ds(0, _SC_TAIL)], agg_sh.at[idxt], add=True)

  plsc.subcore_barrier()

  @pl.when(sid < 15)
  def _():
    r0 = sid * ROWS_A
    pltpu.sync_copy(agg_sh.at[pl.ds(r0, ROWS_A)], aggp.at[cid, pl.ds(r0, ROWS_A)])

  @pl.when(sid == 15)
  def _():
    pltpu.sync_copy(agg_sh.at[pl.ds(15 * ROWS_A, ROWS_B)],
                    aggp.at[cid, pl.ds(15 * ROWS_A, ROWS_B)])


@functools.cache
def _scatter_sc_built():
  return pl.kernel(
      _scatter_body,
      out_type=jax.ShapeDtypeStruct((NC, N, L2), _f32),
      mesh=_mesh(),
      compiler_params=pltpu.CompilerParams(needs_layout_passes=False),
      scratch_types=(
          [pltpu.VMEM((_SC_CH,), _i32),
           pltpu.VMEM((_SC_TAIL,), _i32),
           pltpu.VMEM((_SC_CH, L2), _f32),
           pltpu.VMEM_SHARED((N, L2), _f32)]),
  )


def _scatter_sc(*args):
  return _scatter_sc_built()(*args)


_BN = 2000
_BE = 6400


def _ln(x):
  m = jnp.mean(x, axis=1, keepdims=True)
  x0 = x - m
  v = jnp.mean(x0 * x0, axis=1, keepdims=True)
  return x0 / jnp.sqrt(v + 1e-6)


def _full(shape):
  return pl.BlockSpec(shape, lambda i: (0,) * len(shape))


def _rows(block):
  return pl.BlockSpec(block, lambda i: (i,) + (0,) * (len(block) - 1))


def _node_enc_body(u_r, tag_r, emb_r, we1a_r, we1b_r, be1_r, we2_r, be2_r,
                   w1b_r, w1c_r, b1e_r, h_o, gk_o):
  vel = u_r[...] * DT
  x1 = (vel[:, 0:1] * we1a_r[0:1, :] + vel[:, 1:2] * we1a_r[1:2, :]
        + vel[:, 2:3] * we1a_r[2:3, :])
  m_t = jnp.dot(emb_r[...], we1b_r[...], preferred_element_type=_f32)
  onehot = (tag_r[...] ==
            lax.broadcasted_iota(_i32, (1, N_TYPES), 1).astype(_f32))
  x2 = jnp.dot(onehot.astype(_f32), m_t, preferred_element_type=_f32)
  z = jax.nn.relu(x1 + x2 + be1_r[...])
  h = _ln(jnp.dot(z, we2_r[...], preferred_element_type=_f32) + be2_r[...])
  h_o[...] = h
  gk_o[...] = jnp.concatenate(
      [jnp.dot(h, w1b_r[...], preferred_element_type=_f32) + b1e_r[...],
       jnp.dot(h, w1c_r[...], preferred_element_type=_f32)], axis=1)


def _node_enc(u_t, tag_f, emb, we1a, we1b, be1, we2, be2, w1b, w1c, b1e):
  return pl.pallas_call(
      _node_enc_body,
      grid=(N // _BN,),
      in_specs=[_rows((_BN, 3)), _rows((_BN, 1)), _full((N_TYPES, TYPE_EMB_DIM)),
                _full((3, LATENT)), _full((TYPE_EMB_DIM, LATENT)),
                _full((1, LATENT)), _full((LATENT, LATENT)), _full((1, LATENT)),
                _full((LATENT, LATENT)), _full((LATENT, LATENT)),
                _full((1, LATENT))],
      out_specs=[_rows((_BN, LATENT)), _rows((_BN, L2))],
      out_shape=[jax.ShapeDtypeStruct((N, LATENT), _f32),
                 jax.ShapeDtypeStruct((N, L2), _f32)],
  )(u_t, tag_f, emb, we1a, we1b, be1, we2, be2, w1b, w1c, b1e)


def _edge_enc_mlp_body(dx_r, dy_r, dz_r, wg1_r, bg1_r, wg2_r, bg2_r,
                       hr_r, ks_r, w1a_r, w2_r, b2_r, e_o):
  dx, dy, dz = dx_r[...], dy_r[...], dz_r[...]
  dist = jnp.sqrt(dx * dx + dy * dy + dz * dz + 1e-16)
  feat = (dx * wg1_r[0:1, :] + dy * wg1_r[1:2, :] + dz * wg1_r[2:3, :]
          + dist * wg1_r[3:4, :] + bg1_r[...])
  e = _ln(jnp.dot(jax.nn.relu(feat), wg2_r[...],
                  preferred_element_type=_f32) + bg2_r[...])
  t = jax.nn.relu(
      jnp.dot(e, w1a_r[...], preferred_element_type=_f32)
      + hr_r[:, :LATENT] + ks_r[:, LATENT:])
  en = e + _ln(jnp.dot(t, w2_r[...], preferred_element_type=_f32) + b2_r[...])
  e_o[...] = jnp.concatenate([en, jnp.zeros_like(en)], axis=1)


def _edge_enc_mlp(dx, dy, dz, wg1, bg1, wg2, bg2, h_r, k_s, w1a, w2, b2):
  return pl.pallas_call(
      _edge_enc_mlp_body,
      grid=(E // _BE,),
      in_specs=[_rows((_BE, 1))] * 3
      + [_full((4, LATENT)), _full((1, LATENT)), _full((LATENT, LATENT)),
         _full((1, LATENT))]
      + [_rows((_BE, L2))] * 2
      + [_full((LATENT, LATENT)), _full((LATENT, LATENT)), _full((1, LATENT))],
      out_specs=_rows((_BE, L2)),
      out_shape=jax.ShapeDtypeStruct((E, L2), _f32),
  )(dx, dy, dz, wg1, bg1, wg2, bg2, h_r, k_s, w1a, w2, b2)


def _edge_mlp_body(e_r, hr_r, ks_r, w1a_r, w2_r, b2_r, e_o):
  e = e_r[:, :LATENT]
  t = jax.nn.relu(
      jnp.dot(e, w1a_r[...], preferred_element_type=_f32)
      + hr_r[:, :LATENT] + ks_r[:, LATENT:])
  en = e + _ln(jnp.dot(t, w2_r[...], preferred_element_type=_f32) + b2_r[...])
  e_o[...] = jnp.concatenate([en, jnp.zeros_like(en)], axis=1)


def _edge_mlp(e, h_r, k_s, w1a, w2, b2):
  return pl.pallas_call(
      _edge_mlp_body,
      grid=(E // _BE,),
      in_specs=[_rows((_BE, L2))] * 3
      + [_full((LATENT, LATENT)), _full((LATENT, LATENT)), _full((1, LATENT))],
      out_specs=_rows((_BE, L2)),
      out_shape=jax.ShapeDtypeStruct((E, L2), _f32),
  )(e, h_r, k_s, w1a, w2, b2)


def _node_mlp_prep_body(h_r, a0_r, a1_r, wn1a_r, wn1b_r, bn1_r, wn2_r, bn2_r,
                        w1b_r, w1c_r, b1e_r, h_o, gk_o):
  agg = a0_r[:, :LATENT] + a1_r[:, :LATENT]
  z = jax.nn.relu(
      jnp.dot(h_r[...], wn1a_r[...], preferred_element_type=_f32)
      + jnp.dot(agg, wn1b_r[...], preferred_element_type=_f32) + bn1_r[...])
  hn = h_r[...] + _ln(
      jnp.dot(z, wn2_r[...], preferred_element_type=_f32) + bn2_r[...])
  h_o[...] = hn
  gk_o[...] = jnp.concatenate(
      [jnp.dot(hn, w1b_r[...], preferred_element_type=_f32) + b1e_r[...],
       jnp.dot(hn, w1c_r[...], preferred_element_type=_f32)], axis=1)


def _node_mlp_prep(h, a0, a1, wn1a, wn1b, bn1, wn2, bn2, w1b, w1c, b1e):
  return pl.pallas_call(
      _node_mlp_prep_body,
      grid=(N // _BN,),
      in_specs=[_rows((_BN, LATENT))] + [_rows((_BN, L2))] * 2
      + [_full((LATENT, LATENT))] * 2 + [_full((1, LATENT))]
      + [_full((LATENT, LATENT)), _full((1, LATENT))]
      + [_full((LATENT, LATENT))] * 2 + [_full((1, LATENT))],
      out_specs=[_rows((_BN, LATENT)), _rows((_BN, L2))],
      out_shape=[jax.ShapeDtypeStruct((N, LATENT), _f32),
                 jax.ShapeDtypeStruct((N, L2), _f32)],
  )(h, a0, a1, wn1a, wn1b, bn1, wn2, bn2, w1b, w1c, b1e)


def _node_mlp_dec_body(h_r, a0_r, a1_r, wn1a_r, wn1b_r, bn1_r, wn2_r, bn2_r,
                       wd1_r, bd1_r, wd2_r, bd2_r, d_o):
  agg = a0_r[:, :LATENT] + a1_r[:, :LATENT]
  z = jax.nn.relu(
      jnp.dot(h_r[...], wn1a_r[...], preferred_element_type=_f32)
      + jnp.dot(agg, wn1b_r[...], preferred_element_type=_f32) + bn1_r[...])
  hn = h_r[...] + _ln(
      jnp.dot(z, wn2_r[...], preferred_element_type=_f32) + bn2_r[...])
  td = jax.nn.relu(
      jnp.dot(hn, wd1_r[...], preferred_element_type=_f32) + bd1_r[...])
  d_o[...] = jnp.dot(td, wd2_r[...], preferred_element_type=_f32) + bd2_r[...]


def _node_mlp_dec(h, a0, a1, wn1a, wn1b, bn1, wn2, bn2, wd1, bd1, wd2, bd2):
  return pl.pallas_call(
      _node_mlp_dec_body,
      grid=(N // _BN,),
      in_specs=[_rows((_BN, LATENT))] + [_rows((_BN, L2))] * 2
      + [_full((LATENT, LATENT))] * 2 + [_full((1, LATENT))]
      + [_full((LATENT, LATENT)), _full((1, LATENT))]
      + [_full((LATENT, LATENT)), _full((1, LATENT)), _full((LATENT, 3)),
         _full((1, 3))],
      out_specs=_rows((_BN, 3)),
      out_shape=jax.ShapeDtypeStruct((N, 3), _f32),
  )(h, a0, a1, wn1a, wn1b, bn1, wn2, bn2, wd1, bd1, wd2, bd2)


def _decoder_body(h_r, wd1_r, bd1_r, wd2_r, bd2_r, d_o):
  t = jax.nn.relu(
      jnp.dot(h_r[...], wd1_r[...], preferred_element_type=_f32) + bd1_r[...])
  d_o[...] = jnp.dot(t, wd2_r[...], preferred_element_type=_f32) + bd2_r[...]


def _decoder(h, wd1, bd1, wd2, bd2):
  return pl.pallas_call(
      _decoder_body,
      grid=(N // _BN,),
      in_specs=[_rows((_BN, LATENT)), _full((LATENT, LATENT)),
                _full((1, LATENT)), _full((LATENT, 3)), _full((1, 3))],
      out_specs=_rows((_BN, 3)),
      out_shape=jax.ShapeDtypeStruct((N, 3), _f32),
  )(h, wd1, bd1, wd2, bd2)


def _integrate_body(u_r, r_r, a0_r, a1_r, d_r, r0_r, u0_r, un_o, rn_o, ans_o):
  acc = a0_r[...] + a1_r[...] + d_r[...] * (1.0 / (DT * DT))
  un = u_r[...] + SDT * acc
  rn = r_r[...] + SDT * un
  un_o[...] = un
  rn_o[...] = rn
  ans_o[...] = (rn - r0_r[...]) - u0_r[...] * DT


def _integrate(u3, r3, a0, a1, d_t, r03, u03):
  return pl.pallas_call(
      _integrate_body,
      out_shape=[jax.ShapeDtypeStruct((3, N), _f32)] * 3,
  )(u3, r3, a0, a1, d_t, r03, u03)


def kernel(abs_pos, vel_hist, tag, sph_edge_index, gnn_edge_index, params):
  r3 = abs_pos[:, -1, :].T
  u3 = vel_hist.T * (1.0 / DT)
  ish = sph_edge_index[0]
  jsh = sph_edge_index[1]
  rcv = gnn_edge_index[0]
  snd = gnn_edge_index[1]
  tag_f = tag.astype(_f32)[:, None]

  (we1, be1), (we2, be2) = params["node_enc"]
  emb = params["type_emb"]
  (wg1, bg1), (wg2, bg2) = params["edge_enc"]
  (wd1, bd1), (wd2, bd2) = params["decoder"]
  be1r, be2r = be1[None, :], be2[None, :]
  bg1r, bg2r = bg1[None, :], bg2[None, :]
  bd1r, bd2r = bd1[None, :], bd2[None, :]

  edge_w = []
  for m in range(MP_STEPS):
    (w1, b1), (w2, b2) = params["edge_mlps"][m]
    edge_w.append((w1[:LATENT], w1[LATENT:2 * LATENT], w1[2 * LATENT:],
                   b1[None, :], w2, b2[None, :]))
  node_w = []
  for m in range(MP_STEPS):
    (wn1, bn1), (wn2, bn2) = params["node_mlps"][m]
    node_w.append((wn1[:LATENT], wn1[LATENT:], bn1[None, :], wn2,
                   bn2[None, :]))

  r03, u03 = r3, u3
  zer = jnp.zeros((N, L2), _f32)
  ans = None
  for _ in range(SITL_STEPS):
    (ax0, ay0, az0, ax1, ay1, az1, gx, gy, gz) = _sph_sc(
        r3[0], r3[1], r3[2], u3[0], u3[1], u3[2], ish, jsh, rcv, snd)
    a0 = jnp.stack([ax0, ay0, az0])
    a1 = jnp.stack([ax1, ay1, az1])
    dx = gx[:, None]
    dy = gy[:, None]
    dz = gz[:, None]

    w1a0, w1b0, w1c0, b1e0, _, _ = edge_w[0]
    h, gk_t = _node_enc(u3.T, tag_f, emb, we1[:3], we1[3:], be1r, we2,
                        be2r, w1b0, w1c0, b1e0)

    dcd = None
    for m in range(MP_STEPS):
      w1a, _, _, _, w2, b2r = edge_w[m]
      h_r, k_s = _gather_sc(gk_t, rcv, snd)
      if m == 0:
        e = _edge_enc_mlp(dx, dy, dz, wg1, bg1r, wg2, bg2r, h_r, k_s,
                          w1a, w2, b2r)
      else:
        e = _edge_mlp(e, h_r, k_s, w1a, w2, b2r)
      aggp = _scatter_sc(e, rcv, zer)
      wn1a, wn1b, bn1r, wn2, bn2r = node_w[m]
      if m + 1 < MP_STEPS:
        _, w1b_n, w1c_n, b1e_n, _, _ = edge_w[m + 1]
        h, gk_t = _node_mlp_prep(h, aggp[0], aggp[1], wn1a, wn1b, bn1r,
                                 wn2, bn2r, w1b_n, w1c_n, b1e_n)
      else:
        dcd = _node_mlp_dec(h, aggp[0], aggp[1], wn1a, wn1b, bn1r, wn2,
                            bn2r, wd1, bd1r, wd2, bd2r)

    u3, r3, ans = _integrate(u3, r3, a0, a1, dcd.T, r03, u03)

  return ans.T


# speedup vs baseline: 1.0957x; 1.0590x over previous
import functools
import math

import jax
import jax.numpy as jnp
from jax import lax
from jax.experimental import pallas as pl
from jax.experimental.pallas import tpu as pltpu
from jax.experimental.pallas import tpu_sc as plsc

N = 10000
E = 160000
LATENT = 64
MP_STEPS = 2
SITL_STEPS = 2
DT = 0.002
SDT = DT / SITL_STEPS
SIGMA = 3.0 / (359.0 * math.pi)
PREF = 100.0
ETA_IJ = 2.0 * 0.01 * 0.01 / (0.01 + 0.01 + 1e-08)
GNN_INV_RADIUS = 1.0 / 3.0
N_TYPES = 9
TYPE_EMB_DIM = 16

NC = 2
NS = 16
NW = NC * NS
EPT = E // NW
G2 = EPT + 8
NGRP = G2 // 16
ROWS_A = 640
ROWS_B = N - 15 * ROWS_A

@functools.cache
def _mesh():
  return plsc.VectorSubcoreMesh(
      core_axis_name="c", subcore_axis_name="s", num_cores=NC, num_subcores=NS)


_f32 = jnp.float32
_i32 = jnp.int32


def _sc_sqrt(s):
  i = plsc.bitcast(s, _i32)
  y = plsc.bitcast(jnp.int32(0x1FBD1DF5) + lax.shift_right_arithmetic(i, 1),
                   _f32)
  y = 0.5 * (y + s / y)
  y = 0.5 * (y + s / y)
  y = 0.5 * (y + s / y)
  return y


def _zero_sh_1d(sid, sh, zb):
  @pl.when(sid < 15)
  def _():
    pltpu.sync_copy(zb, sh.at[pl.ds(sid * ROWS_A, ROWS_A)])

  @pl.when(sid == 15)
  def _():
    pltpu.sync_copy(zb.at[pl.ds(0, ROWS_B)], sh.at[pl.ds(15 * ROWS_A, ROWS_B)])


def _sph_body(xp, yp, zp, uxp, uyp, uzp, ish, jsh, rcv, snd,
              ax0, ay0, az0, ax1, ay1, az1, gx, gy, gz,
              xv, yv, zv, uxv, uyv, uzv, rhov, iv, jv, v0, v1, v2, zb,
              rho_sh, ax_sh, ay_sh, az_sh):
  cid = lax.axis_index("c")
  sid = lax.axis_index("s")
  wid = cid * NS + sid

  pltpu.sync_copy(xp, xv)
  pltpu.sync_copy(yp, yv)
  pltpu.sync_copy(zp, zv)
  pltpu.sync_copy(uxp, uxv)
  pltpu.sync_copy(uyp, uyv)
  pltpu.sync_copy(uzp, uzv)

  zi = jnp.full((16,), 0, _i32)
  zf = jnp.full((16,), 0.0, _f32)
  iv[pl.ds(G2 - 16, 16)] = zi
  jv[pl.ds(G2 - 16, 16)] = zi
  for t in range(ROWS_A // 16):
    zb[pl.ds(t * 16, 16)] = zf

  _zero_sh_1d(sid, rho_sh, zb)
  _zero_sh_1d(sid, ax_sh, zb)
  _zero_sh_1d(sid, ay_sh, zb)
  _zero_sh_1d(sid, az_sh, zb)
  plsc.subcore_barrier()

  iota = lax.iota(_i32, 16)

  ebase = wid * EPT
  pltpu.sync_copy(rcv.at[pl.ds(ebase, EPT)], iv.at[pl.ds(0, EPT)])
  pltpu.sync_copy(snd.at[pl.ds(ebase, EPT)], jv.at[pl.ds(0, EPT)])

  def _geom_grp(g, carry):
    off = pl.multiple_of(g * 16, 16)
    ii = iv[pl.ds(off, 16)]
    jj = jv[pl.ds(off, 16)]
    v0[pl.ds(off, 16)] = (plsc.load_gather(xv, [ii])
                          - plsc.load_gather(xv, [jj])) * GNN_INV_RADIUS
    v1[pl.ds(off, 16)] = (plsc.load_gather(yv, [ii])
                          - plsc.load_gather(yv, [jj])) * GNN_INV_RADIUS
    v2[pl.ds(off, 16)] = (plsc.load_gather(zv, [ii])
                          - plsc.load_gather(zv, [jj])) * GNN_INV_RADIUS
    return carry

  lax.fori_loop(0, NGRP, _geom_grp, 0)
  pltpu.sync_copy(v0.at[pl.ds(0, EPT)], gx.at[pl.ds(ebase, EPT)])
  pltpu.sync_copy(v1.at[pl.ds(0, EPT)], gy.at[pl.ds(ebase, EPT)])
  pltpu.sync_copy(v2.at[pl.ds(0, EPT)], gz.at[pl.ds(ebase, EPT)])

  for q in range(2):
    b1 = sid * (2 * EPT) + q * EPT
    pltpu.sync_copy(ish.at[pl.ds(b1, EPT)], iv.at[pl.ds(0, EPT)])
    pltpu.sync_copy(jsh.at[pl.ds(b1, EPT)], jv.at[pl.ds(0, EPT)])

    def _w_grp(g, carry):
      off = pl.multiple_of(g * 16, 16)
      ii = iv[pl.ds(off, 16)]
      jj = jv[pl.ds(off, 16)]
      dx = plsc.load_gather(xv, [ii]) - plsc.load_gather(xv, [jj])
      dy = plsc.load_gather(yv, [ii]) - plsc.load_gather(yv, [jj])
      dz = plsc.load_gather(zv, [ii]) - plsc.load_gather(zv, [jj])
      d = _sc_sqrt(dx * dx + dy * dy + dz * dz + 1e-16)
      q1 = jnp.maximum(0.0, 1.0 - d)
      q2 = jnp.maximum(0.0, 2.0 - d)
      q3 = jnp.maximum(0.0, 3.0 - d)
      q1 = q1 * q1 * q1 * q1 * q1
      q2 = q2 * q2 * q2 * q2 * q2
      q3 = q3 * q3 * q3 * q3 * q3
      w = SIGMA * (q3 - 6.0 * q2 + 15.0 * q1)
      w = jnp.where(off + iota < EPT, w, 0.0)
      v0[pl.ds(off, 16)] = w
      return carry

    lax.fori_loop(0, NGRP, _w_grp, 0)
    pltpu.sync_copy(v0, rho_sh.at[iv], add=True)

  plsc.subcore_barrier()
  pltpu.sync_copy(rho_sh, rhov)

  b2 = wid * EPT
  pltpu.sync_copy(ish.at[pl.ds(b2, EPT)], iv.at[pl.ds(0, EPT)])
  pltpu.sync_copy(jsh.at[pl.ds(b2, EPT)], jv.at[pl.ds(0, EPT)])

  def _a_grp(g, carry):
    off = pl.multiple_of(g * 16, 16)
    ii = iv[pl.ds(off, 16)]
    jj = jv[pl.ds(off, 16)]
    dx = plsc.load_gather(xv, [ii]) - plsc.load_gather(xv, [jj])
    dy = plsc.load_gather(yv, [ii]) - plsc.load_gather(yv, [jj])
    dz = plsc.load_gather(zv, [ii]) - plsc.load_gather(zv, [jj])
    dux = plsc.load_gather(uxv, [ii]) - plsc.load_gather(uxv, [jj])
    duy = plsc.load_gather(uyv, [ii]) - plsc.load_gather(uyv, [jj])
    duz = plsc.load_gather(uzv, [ii]) - plsc.load_gather(uzv, [jj])
    ri = plsc.load_gather(rhov, [ii])
    rj = plsc.load_gather(rhov, [jj])
    d = _sc_sqrt(dx * dx + dy * dy + dz * dz + 1e-16)
    q1 = jnp.maximum(0.0, 1.0 - d)
    q2 = jnp.maximum(0.0, 2.0 - d)
    q3 = jnp.maximum(0.0, 3.0 - d)
    q1 = q1 * q1 * q1 * q1
    q2 = q2 * q2 * q2 * q2
    q3 = q3 * q3 * q3 * q3
    gw = (-5.0 * SIGMA) * (q3 - 6.0 * q2 + 15.0 * q1)
    p_i = PREF * (ri - 1.0)
    p_j = PREF * (rj - 1.0)
    p_ij = (rj * p_i + ri * p_j) / (ri + rj)
    wv = 1.0 / (ri * ri) + 1.0 / (rj * rj)
    c = wv * gw / (d + 1e-08)
    msk = off + iota < EPT
    v0[pl.ds(off, 16)] = jnp.where(
        msk, c * (-p_ij * dx + ETA_IJ * dux), 0.0)
    v1[pl.ds(off, 16)] = jnp.where(
        msk, c * (-p_ij * dy + ETA_IJ * duy), 0.0)
    v2[pl.ds(off, 16)] = jnp.where(
        msk, c * (-p_ij * dz + ETA_IJ * duz), 0.0)
    return carry

  lax.fori_loop(0, NGRP, _a_grp, 0)
  pltpu.sync_copy(v0, ax_sh.at[iv], add=True)
  pltpu.sync_copy(v1, ay_sh.at[iv], add=True)
  pltpu.sync_copy(v2, az_sh.at[iv], add=True)
  plsc.subcore_barrier()

  for sh, o0, o1 in ((ax_sh, ax0, ax1), (ay_sh, ay0, ay1), (az_sh, az0, az1)):
    for core, out in ((0, o0), (1, o1)):
      @pl.when((cid == core) & (sid < 15))
      def _():
        r0 = sid * ROWS_A
        pltpu.sync_copy(sh.at[pl.ds(r0, ROWS_A)], out.at[pl.ds(r0, ROWS_A)])

      @pl.when((cid == core) & (sid == 15))
      def _():
        pltpu.sync_copy(sh.at[pl.ds(15 * ROWS_A, ROWS_B)],
                        zb.at[pl.ds(0, ROWS_B)])
        pltpu.sync_copy(zb.at[pl.ds(0, ROWS_B)],
                        out.at[pl.ds(15 * ROWS_A, ROWS_B)])


@functools.cache
def _sph_sc_built():
  return pl.kernel(
      _sph_body,
      out_type=tuple([jax.ShapeDtypeStruct((N,), _f32)] * 6
                     + [jax.ShapeDtypeStruct((E,), _f32)] * 3),
      mesh=_mesh(),
      compiler_params=pltpu.CompilerParams(needs_layout_passes=False),
      scratch_types=(
          [pltpu.VMEM((N,), _f32)] * 7
          + [pltpu.VMEM((G2,), _i32)] * 2
          + [pltpu.VMEM((G2,), _f32)] * 3
          + [pltpu.VMEM((ROWS_A,), _f32)]
          + [pltpu.VMEM_SHARED((N,), _f32)] * 4),
  )


def _sph_sc(*args):
  return _sph_sc_built()(*args)


_GC = 1000
L2 = 2 * LATENT
_bf16 = jnp.bfloat16


def _gather_body(gk_t, rcv, snd, out_r, out_s, idxr, idxs, rows):
  cid = lax.axis_index("c")
  sid = lax.axis_index("s")
  wid = cid * NS + sid
  for ch in range(EPT // _GC):
    eb = wid * EPT + ch * _GC
    pltpu.sync_copy(rcv.at[pl.ds(eb, _GC)], idxr)
    pltpu.sync_copy(snd.at[pl.ds(eb, _GC)], idxs)
    pltpu.sync_copy(gk_t.at[idxr], rows)
    pltpu.sync_copy(rows, out_r.at[pl.ds(eb, _GC)])
    pltpu.sync_copy(gk_t.at[idxs], rows)
    pltpu.sync_copy(rows, out_s.at[pl.ds(eb, _GC)])


@functools.cache
def _gather_sc_built():
  return pl.kernel(
      _gather_body,
      out_type=(jax.ShapeDtypeStruct((E, L2), _f32),
                jax.ShapeDtypeStruct((E, L2), _f32)),
      mesh=_mesh(),
      compiler_params=pltpu.CompilerParams(needs_layout_passes=False),
      scratch_types=(
          [pltpu.VMEM((_GC,), _i32)] * 2
          + [pltpu.VMEM((_GC, L2), _f32)]),
  )


def _gather_sc(*args):
  return _gather_sc_built()(*args)


_SC_CH = 96
_SC_TAIL = EPT - (EPT // _SC_CH) * _SC_CH
_SC_NCH = EPT // _SC_CH


def _scatter_body(e_t, rcv, zer, aggp, idx_a, idx_b, idxt, rows_a, rows_b,
                  sem_a, sem_b, agg_sh):
  cid = lax.axis_index("c")
  sid = lax.axis_index("s")
  wid = cid * NS + sid

  @pl.when(sid < 15)
  def _():
    r0 = sid * ROWS_A
    pltpu.sync_copy(zer.at[pl.ds(r0, ROWS_A)], agg_sh.at[pl.ds(r0, ROWS_A)])

  @pl.when(sid == 15)
  def _():
    pltpu.sync_copy(zer.at[pl.ds(15 * ROWS_A, ROWS_B)],
                    agg_sh.at[pl.ds(15 * ROWS_A, ROWS_B)])

  plsc.subcore_barrier()

  idxs = (idx_a, idx_b)
  rows = (rows_a, rows_b)
  sems = (sem_a, sem_b)

  def _start(ch, b):
    eb = wid * EPT + ch * _SC_CH
    d1 = pltpu.async_copy(rcv.at[pl.ds(eb, _SC_CH)], idxs[b], sems[b])
    d2 = pltpu.async_copy(e_t.at[pl.ds(eb, _SC_CH)], rows[b], sems[b])
    return d1, d2

  cur = _start(0, 0)
  for ch in range(_SC_NCH):
    nxt = _start(ch + 1, (ch + 1) % 2) if ch + 1 < _SC_NCH else None
    cur[0].wait()
    cur[1].wait()
    b = ch % 2
    pltpu.sync_copy(rows[b], agg_sh.at[idxs[b]], add=True)
    cur = nxt

  tb = wid * EPT + _SC_NCH * _SC_CH
  pltpu.sync_copy(rcv.at[pl.ds(tb, _SC_TAIL)], idxt)
  pltpu.sync_copy(e_t.at[pl.ds(tb, _SC_TAIL)], rows_a.at[pl.ds(0, _SC_TAIL)])
  pltpu.sync_copy(rows_a.at[pl.ds(0, _SC_TAIL)], agg_sh.at[idxt], add=True)

  plsc.subcore_barrier()

  @pl.when(sid < 15)
  def _():
    r0 = sid * ROWS_A
    pltpu.sync_copy(agg_sh.at[pl.ds(r0, ROWS_A)], aggp.at[cid, pl.ds(r0, ROWS_A)])

  @pl.when(sid == 15)
  def _():
    pltpu.sync_copy(agg_sh.at[pl.ds(15 * ROWS_A, ROWS_B)],
                    aggp.at[cid, pl.ds(15 * ROWS_A, ROWS_B)])


@functools.cache
def _scatter_sc_built():
  return pl.kernel(
      _scatter_body,
      out_type=jax.ShapeDtypeStruct((NC, N, L2), _f32),
      mesh=_mesh(),
      compiler_params=pltpu.CompilerParams(needs_layout_passes=False),
      scratch_types=(
          [pltpu.VMEM((_SC_CH,), _i32),
           pltpu.VMEM((_SC_CH,), _i32),
           pltpu.VMEM((_SC_TAIL,), _i32),
           pltpu.VMEM((_SC_CH, L2), _f32),
           pltpu.VMEM((_SC_CH, L2), _f32),
           pltpu.SemaphoreType.DMA,
           pltpu.SemaphoreType.DMA,
           pltpu.VMEM_SHARED((N, L2), _f32)]),
  )


def _scatter_sc(*args):
  return _scatter_sc_built()(*args)


_BN = 2000
_BE = 6400


def _ln(x):
  m = jnp.mean(x, axis=1, keepdims=True)
  x0 = x - m
  v = jnp.mean(x0 * x0, axis=1, keepdims=True)
  return x0 / jnp.sqrt(v + 1e-6)


def _full(shape):
  return pl.BlockSpec(shape, lambda i: (0,) * len(shape))


def _rows(block):
  return pl.BlockSpec(block, lambda i: (i,) + (0,) * (len(block) - 1))


def _node_enc_body(u_r, tag_r, emb_r, we1a_r, we1b_r, be1_r, we2_r, be2_r,
                   w1b_r, w1c_r, b1e_r, h_o, gk_o):
  vel = u_r[...] * DT
  x1 = (vel[:, 0:1] * we1a_r[0:1, :] + vel[:, 1:2] * we1a_r[1:2, :]
        + vel[:, 2:3] * we1a_r[2:3, :])
  m_t = jnp.dot(emb_r[...], we1b_r[...], preferred_element_type=_f32)
  onehot = (tag_r[...] ==
            lax.broadcasted_iota(_i32, (1, N_TYPES), 1).astype(_f32))
  x2 = jnp.dot(onehot.astype(_f32), m_t, preferred_element_type=_f32)
  z = jax.nn.relu(x1 + x2 + be1_r[...])
  h = _ln(jnp.dot(z, we2_r[...], preferred_element_type=_f32) + be2_r[...])
  h_o[...] = h
  gk_o[...] = jnp.concatenate(
      [jnp.dot(h, w1b_r[...], preferred_element_type=_f32) + b1e_r[...],
       jnp.dot(h, w1c_r[...], preferred_element_type=_f32)], axis=1)


def _node_enc(u_t, tag_f, emb, we1a, we1b, be1, we2, be2, w1b, w1c, b1e):
  return pl.pallas_call(
      _node_enc_body,
      grid=(N // _BN,),
      in_specs=[_rows((_BN, 3)), _rows((_BN, 1)), _full((N_TYPES, TYPE_EMB_DIM)),
                _full((3, LATENT)), _full((TYPE_EMB_DIM, LATENT)),
                _full((1, LATENT)), _full((LATENT, LATENT)), _full((1, LATENT)),
                _full((LATENT, LATENT)), _full((LATENT, LATENT)),
                _full((1, LATENT))],
      out_specs=[_rows((_BN, LATENT)), _rows((_BN, L2))],
      out_shape=[jax.ShapeDtypeStruct((N, LATENT), _f32),
                 jax.ShapeDtypeStruct((N, L2), _f32)],
  )(u_t, tag_f, emb, we1a, we1b, be1, we2, be2, w1b, w1c, b1e)


def _edge_enc_mlp_body(dx_r, dy_r, dz_r, wg1_r, bg1_r, wg2_r, bg2_r,
                       hr_r, ks_r, w1a_r, w2_r, b2_r, e_o):
  dx, dy, dz = dx_r[...], dy_r[...], dz_r[...]
  dist = jnp.sqrt(dx * dx + dy * dy + dz * dz + 1e-16)
  feat = (dx * wg1_r[0:1, :] + dy * wg1_r[1:2, :] + dz * wg1_r[2:3, :]
          + dist * wg1_r[3:4, :] + bg1_r[...])
  e = _ln(jnp.dot(jax.nn.relu(feat), wg2_r[...],
                  preferred_element_type=_f32) + bg2_r[...])
  t = jax.nn.relu(
      jnp.dot(e, w1a_r[...], preferred_element_type=_f32)
      + hr_r[:, :LATENT] + ks_r[:, LATENT:])
  en = e + _ln(jnp.dot(t, w2_r[...], preferred_element_type=_f32) + b2_r[...])
  e_o[...] = jnp.concatenate([en, jnp.zeros_like(en)], axis=1)


def _edge_enc_mlp(dx, dy, dz, wg1, bg1, wg2, bg2, h_r, k_s, w1a, w2, b2):
  return pl.pallas_call(
      _edge_enc_mlp_body,
      grid=(E // _BE,),
      in_specs=[_rows((_BE, 1))] * 3
      + [_full((4, LATENT)), _full((1, LATENT)), _full((LATENT, LATENT)),
         _full((1, LATENT))]
      + [_rows((_BE, L2))] * 2
      + [_full((LATENT, LATENT)), _full((LATENT, LATENT)), _full((1, LATENT))],
      out_specs=_rows((_BE, L2)),
      out_shape=jax.ShapeDtypeStruct((E, L2), _f32),
  )(dx, dy, dz, wg1, bg1, wg2, bg2, h_r, k_s, w1a, w2, b2)


def _edge_mlp_body(e_r, hr_r, ks_r, w1a_r, w2_r, b2_r, e_o):
  e = e_r[:, :LATENT]
  t = jax.nn.relu(
      jnp.dot(e, w1a_r[...], preferred_element_type=_f32)
      + hr_r[:, :LATENT] + ks_r[:, LATENT:])
  en = e + _ln(jnp.dot(t, w2_r[...], preferred_element_type=_f32) + b2_r[...])
  e_o[...] = jnp.concatenate([en, jnp.zeros_like(en)], axis=1)


def _edge_mlp(e, h_r, k_s, w1a, w2, b2):
  return pl.pallas_call(
      _edge_mlp_body,
      grid=(E // _BE,),
      in_specs=[_rows((_BE, L2))] * 3
      + [_full((LATENT, LATENT)), _full((LATENT, LATENT)), _full((1, LATENT))],
      out_specs=_rows((_BE, L2)),
      out_shape=jax.ShapeDtypeStruct((E, L2), _f32),
  )(e, h_r, k_s, w1a, w2, b2)


def _node_mlp_prep_body(h_r, a0_r, a1_r, wn1a_r, wn1b_r, bn1_r, wn2_r, bn2_r,
                        w1b_r, w1c_r, b1e_r, h_o, gk_o):
  agg = a0_r[:, :LATENT] + a1_r[:, :LATENT]
  z = jax.nn.relu(
      jnp.dot(h_r[...], wn1a_r[...], preferred_element_type=_f32)
      + jnp.dot(agg, wn1b_r[...], preferred_element_type=_f32) + bn1_r[...])
  hn = h_r[...] + _ln(
      jnp.dot(z, wn2_r[...], preferred_element_type=_f32) + bn2_r[...])
  h_o[...] = hn
  gk_o[...] = jnp.concatenate(
      [jnp.dot(hn, w1b_r[...], preferred_element_type=_f32) + b1e_r[...],
       jnp.dot(hn, w1c_r[...], preferred_element_type=_f32)], axis=1)


def _node_mlp_prep(h, a0, a1, wn1a, wn1b, bn1, wn2, bn2, w1b, w1c, b1e):
  return pl.pallas_call(
      _node_mlp_prep_body,
      grid=(N // _BN,),
      in_specs=[_rows((_BN, LATENT))] + [_rows((_BN, L2))] * 2
      + [_full((LATENT, LATENT))] * 2 + [_full((1, LATENT))]
      + [_full((LATENT, LATENT)), _full((1, LATENT))]
      + [_full((LATENT, LATENT))] * 2 + [_full((1, LATENT))],
      out_specs=[_rows((_BN, LATENT)), _rows((_BN, L2))],
      out_shape=[jax.ShapeDtypeStruct((N, LATENT), _f32),
                 jax.ShapeDtypeStruct((N, L2), _f32)],
  )(h, a0, a1, wn1a, wn1b, bn1, wn2, bn2, w1b, w1c, b1e)


def _node_mlp_dec_body(h_r, a0_r, a1_r, wn1a_r, wn1b_r, bn1_r, wn2_r, bn2_r,
                       wd1_r, bd1_r, wd2_r, bd2_r, d_o):
  agg = a0_r[:, :LATENT] + a1_r[:, :LATENT]
  z = jax.nn.relu(
      jnp.dot(h_r[...], wn1a_r[...], preferred_element_type=_f32)
      + jnp.dot(agg, wn1b_r[...], preferred_element_type=_f32) + bn1_r[...])
  hn = h_r[...] + _ln(
      jnp.dot(z, wn2_r[...], preferred_element_type=_f32) + bn2_r[...])
  td = jax.nn.relu(
      jnp.dot(hn, wd1_r[...], preferred_element_type=_f32) + bd1_r[...])
  d_o[...] = jnp.dot(td, wd2_r[...], preferred_element_type=_f32) + bd2_r[...]


def _node_mlp_dec(h, a0, a1, wn1a, wn1b, bn1, wn2, bn2, wd1, bd1, wd2, bd2):
  return pl.pallas_call(
      _node_mlp_dec_body,
      grid=(N // _BN,),
      in_specs=[_rows((_BN, LATENT))] + [_rows((_BN, L2))] * 2
      + [_full((LATENT, LATENT))] * 2 + [_full((1, LATENT))]
      + [_full((LATENT, LATENT)), _full((1, LATENT))]
      + [_full((LATENT, LATENT)), _full((1, LATENT)), _full((LATENT, 3)),
         _full((1, 3))],
      out_specs=_rows((_BN, 3)),
      out_shape=jax.ShapeDtypeStruct((N, 3), _f32),
  )(h, a0, a1, wn1a, wn1b, bn1, wn2, bn2, wd1, bd1, wd2, bd2)


def _decoder_body(h_r, wd1_r, bd1_r, wd2_r, bd2_r, d_o):
  t = jax.nn.relu(
      jnp.dot(h_r[...], wd1_r[...], preferred_element_type=_f32) + bd1_r[...])
  d_o[...] = jnp.dot(t, wd2_r[...], preferred_element_type=_f32) + bd2_r[...]


def _decoder(h, wd1, bd1, wd2, bd2):
  return pl.pallas_call(
      _decoder_body,
      grid=(N // _BN,),
      in_specs=[_rows((_BN, LATENT)), _full((LATENT, LATENT)),
                _full((1, LATENT)), _full((LATENT, 3)), _full((1, 3))],
      out_specs=_rows((_BN, 3)),
      out_shape=jax.ShapeDtypeStruct((N, 3), _f32),
  )(h, wd1, bd1, wd2, bd2)


def _integrate_body(u_r, r_r, a0_r, a1_r, d_r, r0_r, u0_r, un_o, rn_o, ans_o):
  acc = a0_r[...] + a1_r[...] + d_r[...] * (1.0 / (DT * DT))
  un = u_r[...] + SDT * acc
  rn = r_r[...] + SDT * un
  un_o[...] = un
  rn_o[...] = rn
  ans_o[...] = (rn - r0_r[...]) - u0_r[...] * DT


def _integrate(u3, r3, a0, a1, d_t, r03, u03):
  return pl.pallas_call(
      _integrate_body,
      out_shape=[jax.ShapeDtypeStruct((3, N), _f32)] * 3,
  )(u3, r3, a0, a1, d_t, r03, u03)


def kernel(abs_pos, vel_hist, tag, sph_edge_index, gnn_edge_index, params):
  r3 = abs_pos[:, -1, :].T
  u3 = vel_hist.T * (1.0 / DT)
  ish = sph_edge_index[0]
  jsh = sph_edge_index[1]
  rcv = gnn_edge_index[0]
  snd = gnn_edge_index[1]
  tag_f = tag.astype(_f32)[:, None]

  (we1, be1), (we2, be2) = params["node_enc"]
  emb = params["type_emb"]
  (wg1, bg1), (wg2, bg2) = params["edge_enc"]
  (wd1, bd1), (wd2, bd2) = params["decoder"]
  be1r, be2r = be1[None, :], be2[None, :]
  bg1r, bg2r = bg1[None, :], bg2[None, :]
  bd1r, bd2r = bd1[None, :], bd2[None, :]

  edge_w = []
  for m in range(MP_STEPS):
    (w1, b1), (w2, b2) = params["edge_mlps"][m]
    edge_w.append((w1[:LATENT], w1[LATENT:2 * LATENT], w1[2 * LATENT:],
                   b1[None, :], w2, b2[None, :]))
  node_w = []
  for m in range(MP_STEPS):
    (wn1, bn1), (wn2, bn2) = params["node_mlps"][m]
    node_w.append((wn1[:LATENT], wn1[LATENT:], bn1[None, :], wn2,
                   bn2[None, :]))

  r03, u03 = r3, u3
  zer = jnp.zeros((N, L2), _f32)
  ans = None
  for _ in range(SITL_STEPS):
    (ax0, ay0, az0, ax1, ay1, az1, gx, gy, gz) = _sph_sc(
        r3[0], r3[1], r3[2], u3[0], u3[1], u3[2], ish, jsh, rcv, snd)
    a0 = jnp.stack([ax0, ay0, az0])
    a1 = jnp.stack([ax1, ay1, az1])
    dx = gx[:, None]
    dy = gy[:, None]
    dz = gz[:, None]

    w1a0, w1b0, w1c0, b1e0, _, _ = edge_w[0]
    h, gk_t = _node_enc(u3.T, tag_f, emb, we1[:3], we1[3:], be1r, we2,
                        be2r, w1b0, w1c0, b1e0)

    dcd = None
    for m in range(MP_STEPS):
      w1a, _, _, _, w2, b2r = edge_w[m]
      h_r, k_s = _gather_sc(gk_t, rcv, snd)
      if m == 0:
        e = _edge_enc_mlp(dx, dy, dz, wg1, bg1r, wg2, bg2r, h_r, k_s,
                          w1a, w2, b2r)
      else:
        e = _edge_mlp(e, h_r, k_s, w1a, w2, b2r)
      aggp = _scatter_sc(e, rcv, zer)
      wn1a, wn1b, bn1r, wn2, bn2r = node_w[m]
      if m + 1 < MP_STEPS:
        _, w1b_n, w1c_n, b1e_n, _, _ = edge_w[m + 1]
        h, gk_t = _node_mlp_prep(h, aggp[0], aggp[1], wn1a, wn1b, bn1r,
                                 wn2, bn2r, w1b_n, w1c_n, b1e_n)
      else:
        dcd = _node_mlp_dec(h, aggp[0], aggp[1], wn1a, wn1b, bn1r, wn2,
                            bn2r, wd1, bd1r, wd2, bd2r)

    u3, r3, ans = _integrate(u3, r3, a0, a1, dcd.T, r03, u03)

  return ans.T
